# baseline (speedup 1.0000x reference)
import functools

import jax
import jax.numpy as jnp
from jax import lax
from jax.experimental import pallas as pl
from jax.experimental.pallas import tpu as pltpu
from jax.experimental.pallas import tpu_sc as plsc

BATCH = 4096
DIM = 128
BLK = 1024
NBLK = BATCH // BLK
NSTEPS = NBLK * (NBLK + 1) // 2
_OFFS = [0, 4, 7, 9]

_NC, _NS, _NL = 1, 16, 16
_NW = _NC * _NS
_LPW = BATCH // _NW
_HBINS = 1024


def _tri_ij(t):
    i = ((t >= _OFFS[1]).astype(jnp.int32)
         + (t >= _OFFS[2]).astype(jnp.int32)
         + (t >= _OFFS[3]).astype(jnp.int32))
    off = ((t >= _OFFS[1]).astype(jnp.int32) * (_OFFS[1] - _OFFS[0])
           + (t >= _OFFS[2]).astype(jnp.int32) * (_OFFS[2] - _OFFS[1])
           + (t >= _OFFS[3]).astype(jnp.int32) * (_OFFS[3] - _OFFS[2]))
    j = t - off + i
    return i, j


def _loss_body(a_ref, b_ref, lab_i_ref, lab_j_ref, out_ref, acc_ref):
    t = pl.program_id(0)
    i, j = _tri_ij(t)

    @pl.when(t == 0)
    def _init():
        acc_ref[...] = jnp.zeros(acc_ref.shape, acc_ref.dtype)

    a = a_ref[...]
    b = b_ref[...]
    g2 = jax.lax.dot_general(
        a * (-2.0), b, dimension_numbers=(((1,), (1,)), ((), ())),
        preferred_element_type=jnp.float32)
    an = jnp.sum(a * a, axis=1, keepdims=True)
    bn = jnp.sum(b * b, axis=1)[None, :]
    sq = (an + bn) + g2
    sq_bf = sq.astype(jnp.bfloat16)
    eps = jnp.bfloat16(1e-16)
    m = jnp.where(sq_bf > eps, sq_bf, eps)
    dist_bf = m * jax.lax.rsqrt(m)

    pos_m = lab_i_ref[...] == lab_j_ref[...]
    pos_fb = jnp.where(pos_m, 1.0, 0.0).astype(jnp.bfloat16)
    omfb = jnp.bfloat16(1.0) - pos_fb
    zero = jnp.zeros((), jnp.bfloat16)
    tneg_bf = jnp.bfloat16(1.0) - dist_bf
    trm16 = tneg_bf > zero
    u = tneg_bf * omfb
    r1 = dist_bf * pos_fb
    r3 = jnp.where(trm16, u, zero)
    r4 = jnp.where(trm16, omfb, zero)

    ones_row = jnp.ones((1, BLK), jnp.bfloat16)

    def colsum(x):
        return jax.lax.dot_general(
            ones_row, x, dimension_numbers=(((1,), (0,)), ((), ())),
            preferred_element_type=jnp.float32)

    w = jnp.where(i == j, 1.0, 2.0)
    acc_ref[0:1, :] = acc_ref[0:1, :] + w * colsum(r1)
    acc_ref[2:3, :] = acc_ref[2:3, :] + w * colsum(r3)
    acc_ref[3:4, :] = acc_ref[3:4, :] + w * colsum(r4)

    @pl.when(t == NSTEPS - 1)
    def _fini():
        out_ref[0] = jnp.sum(acc_ref[0:1, :])
        out_ref[1] = jnp.sum(acc_ref[2:3, :])
        out_ref[2] = jnp.sum(acc_ref[3:4, :])


def _tc_sums(emb, lab_col, lab_row):
    return pl.pallas_call(
        _loss_body,
        grid=(NSTEPS,),
        in_specs=[
            pl.BlockSpec((BLK, DIM), lambda t: (_tri_ij(t)[0], 0)),
            pl.BlockSpec((BLK, DIM), lambda t: (_tri_ij(t)[1], 0)),
            pl.BlockSpec((BLK, 1), lambda t: (_tri_ij(t)[0], 0)),
            pl.BlockSpec((1, BLK), lambda t: (0, _tri_ij(t)[1])),
        ],
        out_specs=pl.BlockSpec(memory_space=pltpu.SMEM),
        out_shape=jax.ShapeDtypeStruct((4,), jnp.float32),
        scratch_shapes=[pltpu.VMEM((8, BLK), jnp.float32)],
    )(emb, emb, lab_col, lab_row)


_sc_mesh = plsc.VectorSubcoreMesh(core_axis_name="c", subcore_axis_name="s", num_cores=1)


@functools.partial(
    pl.kernel,
    out_type=jax.ShapeDtypeStruct((_NL,), jnp.float32),
    mesh=_sc_mesh,
    scratch_types=[
        pltpu.VMEM((_LPW,), jnp.int32),
        pltpu.VMEM((_LPW,), jnp.float32),
        pltpu.VMEM((_HBINS,), jnp.float32),
        pltpu.VMEM_SHARED((_HBINS,), jnp.float32),
        pltpu.VMEM((_NL,), jnp.float32),
    ],
)
def _pos_cnt_sc(lab_hbm, out_hbm, idx_v, ones_v, hist_v, shared_h, acc_v):
    wid = lax.axis_index("s") * _NC + lax.axis_index("c")

    for k in range(_LPW // _NL):
        ones_v[pl.ds(k * _NL, _NL)] = jnp.ones((_NL,), jnp.float32)
    for k in range(_HBINS // _NL):
        hist_v[pl.ds(k * _NL, _NL)] = jnp.zeros((_NL,), jnp.float32)

    @pl.when(wid == 0)
    def _zero_shared():
        pltpu.sync_copy(hist_v, shared_h)

    plsc.subcore_barrier()
    pltpu.sync_copy(lab_hbm.at[pl.ds(wid * _LPW, _LPW)], idx_v)
    pltpu.sync_copy(ones_v, shared_h.at[idx_v], add=True)
    plsc.subcore_barrier()

    @pl.when(wid == 0)
    def _square_sum():
        pltpu.sync_copy(shared_h, hist_v)
        acc_v[...] = jnp.zeros((_NL,), jnp.float32)
        for k in range(_HBINS // _NL):
            h = hist_v[pl.ds(k * _NL, _NL)]
            acc_v[...] = acc_v[...] + h * h
        pltpu.sync_copy(acc_v, out_hbm)


def kernel(embeddings, labels):
    emb = embeddings.astype(jnp.float32)
    lab = labels.astype(jnp.int32)
    lab_col = lab.reshape(BATCH, 1)
    lab_row = lab.reshape(1, BATCH)
    sums = _tc_sums(emb, lab_col, lab_row)
    pos_cnt = jnp.sum(_pos_cnt_sc(lab))
    pos_avg = sums[0] / jnp.maximum(pos_cnt, 1.0)
    neg_avg = sums[1] / jnp.maximum(sums[2], 1.0)
    return pos_avg + neg_avg


# speedup vs baseline: 1.5547x; 1.5547x over previous
import jax
import jax.numpy as jnp
from jax.experimental import pallas as pl
from jax.experimental.pallas import tpu as pltpu

BATCH = 4096
DIM = 128
BLK = 1024
NBLK = BATCH // BLK
NSTEPS = NBLK * (NBLK + 1) // 2
HBINS = 1024
_OFFS = [0, 4, 7, 9]


def _tri_ij(t):
    i = ((t >= _OFFS[1]).astype(jnp.int32)
         + (t >= _OFFS[2]).astype(jnp.int32)
         + (t >= _OFFS[3]).astype(jnp.int32))
    off = ((t >= _OFFS[1]).astype(jnp.int32) * (_OFFS[1] - _OFFS[0])
           + (t >= _OFFS[2]).astype(jnp.int32) * (_OFFS[2] - _OFFS[1])
           + (t >= _OFFS[3]).astype(jnp.int32) * (_OFFS[3] - _OFFS[2]))
    j = t - off + i
    return i, j


def _loss_body(a_ref, b_ref, lab_i_ref, lab_j_ref, lab_full_ref, out_ref,
               acc_ref):
    t = pl.program_id(0)
    i, j = _tri_ij(t)

    @pl.when(t == 0)
    def _init():
        acc_ref[...] = jnp.zeros(acc_ref.shape, acc_ref.dtype)

    a = a_ref[...]
    b = b_ref[...]
    g2 = jax.lax.dot_general(
        a * (-2.0), b, dimension_numbers=(((1,), (1,)), ((), ())),
        preferred_element_type=jnp.float32)
    an = jnp.sum(a * a, axis=1, keepdims=True)
    bn = jnp.sum(b * b, axis=1)[None, :]
    sq = (an + bn) + g2
    sq_bf = sq.astype(jnp.bfloat16)
    eps = jnp.bfloat16(1e-16)
    m = jnp.where(sq_bf > eps, sq_bf, eps)
    dist_bf = m * jax.lax.rsqrt(m)

    pos_m = lab_i_ref[...] == lab_j_ref[...]
    pos_fb = jnp.where(pos_m, 1.0, 0.0).astype(jnp.bfloat16)
    omfb = jnp.bfloat16(1.0) - pos_fb
    one = jnp.ones((), jnp.bfloat16)
    zero = jnp.zeros((), jnp.bfloat16)
    tneg_bf = one - dist_bf
    trm16 = tneg_bf > zero
    u = tneg_bf * omfb
    r1 = dist_bf * pos_fb
    r3 = jnp.where(trm16, u, zero)
    r4 = jnp.where(trm16, omfb, zero)

    ones_row = jnp.ones((1, BLK), jnp.bfloat16)

    def colsum(x):
        return jax.lax.dot_general(
            ones_row, x, dimension_numbers=(((1,), (0,)), ((), ())),
            preferred_element_type=jnp.float32)

    w = jnp.where(i == j, 1.0, 2.0)
    acc_ref[0:1, :] = acc_ref[0:1, :] + w * colsum(r1)
    acc_ref[2:3, :] = acc_ref[2:3, :] + w * colsum(r3)
    acc_ref[3:4, :] = acc_ref[3:4, :] + w * colsum(r4)

    @pl.when(t == NSTEPS - 1)
    def _fini():
        iota_col = jax.lax.broadcasted_iota(jnp.int32, (HBINS, 1), 0)
        onehot = jnp.where(iota_col == lab_full_ref[...], 1.0,
                           0.0).astype(jnp.bfloat16)
        ones_col = jnp.ones((BATCH, 1), jnp.bfloat16)
        h = jax.lax.dot_general(
            onehot, ones_col, dimension_numbers=(((1,), (0,)), ((), ())),
            preferred_element_type=jnp.float32)
        pos_cnt = jnp.sum(h * h)
        pos_sum = jnp.sum(acc_ref[0:1, :])
        neg_sum = jnp.sum(acc_ref[2:3, :])
        neg_cnt = jnp.sum(acc_ref[3:4, :])
        pos_avg = pos_sum / jnp.maximum(pos_cnt, 1.0)
        neg_avg = neg_sum / jnp.maximum(neg_cnt, 1.0)
        out_ref[...] = jnp.reshape(pos_avg + neg_avg, (1, 1))


def kernel(embeddings, labels):
    emb = embeddings.astype(jnp.float32)
    lab = labels.astype(jnp.int32)
    lab_col = lab.reshape(BATCH, 1)
    lab_row = lab.reshape(1, BATCH)
    out = pl.pallas_call(
        _loss_body,
        grid=(NSTEPS,),
        in_specs=[
            pl.BlockSpec((BLK, DIM), lambda t: (_tri_ij(t)[0], 0)),
            pl.BlockSpec((BLK, DIM), lambda t: (_tri_ij(t)[1], 0)),
            pl.BlockSpec((BLK, 1), lambda t: (_tri_ij(t)[0], 0)),
            pl.BlockSpec((1, BLK), lambda t: (0, _tri_ij(t)[1])),
            pl.BlockSpec((1, BATCH), lambda t: (0, 0)),
        ],
        out_specs=pl.BlockSpec((1, 1), lambda t: (0, 0)),
        out_shape=jax.ShapeDtypeStruct((1, 1), jnp.float32),
        scratch_shapes=[pltpu.VMEM((8, BLK), jnp.float32)],
    )(emb, emb, lab_col, lab_row, lab_row)
    return out[0, 0]
